# baseline (speedup 1.0000x reference)
import functools

import jax
import jax.numpy as jnp
from jax import lax
from jax.experimental import pallas as pl
from jax.experimental.pallas import tpu as pltpu
from jax.experimental.pallas import tpu_sc as plsc

B, C, H, W = 2, 256, 320, 320
N = 512
NV = B * N
NWORK = 32
VPW = NV // NWORK
RPW = 2 * VPW
NROWS = NV * (C // 128)


def _sc_gather(fm_rows, pos2d):
    mesh = plsc.VectorSubcoreMesh(core_axis_name="c", subcore_axis_name="s")

    @functools.partial(
        pl.kernel,
        out_type=jax.ShapeDtypeStruct((NROWS, 128), jnp.float32),
        mesh=mesh,
        scratch_types=[
            pltpu.VMEM((160,), jnp.int32),
            pltpu.VMEM((RPW,), jnp.int32),
            pltpu.VMEM((RPW, 128), jnp.float32),
            pltpu.SemaphoreType.DMA,
            pltpu.SemaphoreType.DMA,
            pltpu.SemaphoreType.DMA,
        ],
        compiler_params=pltpu.CompilerParams(
            needs_layout_passes=False,
            skip_device_barrier=True,
            disable_bounds_checks=True,
            disable_semaphore_checks=True,
        ),
    )
    def body(fm_hbm, pos_hbm, out_hbm, pos_v, idx_v, dat_v, gsem0, gsem1,
             osem):
        wid = lax.axis_index("s") * 2 + lax.axis_index("c")
        v0 = wid * VPW
        bq = lax.shift_right_logical(v0, 9)
        k0 = lax.shift_right_logical(v0 & 511, 7)
        o = v0 & 127
        base = pl.multiple_of((bq * 8 + k0 * 2) * 128 + o, VPW)
        pltpu.sync_copy(pos_hbm.at[pl.ds(base, 160)], pos_v)

        lane = jax.lax.iota(jnp.int32, 16)
        cg = RPW // 4
        sems = (gsem0, gsem1)
        gathers = []
        for vc in range(VPW // 16):
            v_loc = vc * 16 + lane
            r = pos_v[pl.ds(vc * 16, 16)]
            c = pos_v[pl.ds(128 + vc * 16, 16)]
            b = lax.shift_right_logical(v0 + vc * 16 + lane, 9)
            rho0 = ((b * H + r) * (W // 8) + lax.shift_right_logical(c, 3)) \
                * 16 + (c & 7)
            pos = v_loc * 2
            plsc.store_scatter(idx_v, [pos], rho0)
            plsc.store_scatter(idx_v, [pos + 1], rho0 + 8)
            for half in range(2):
                j = vc * 2 + half
                gathers.append(
                    pltpu.async_copy(fm_hbm.at[idx_v.at[pl.ds(j * cg, cg)]],
                                     dat_v.at[pl.ds(j * cg, cg)],
                                     sems[j & 1]))

        outs = []
        for j in range(4):
            gathers[j].wait()
            outs.append(
                pltpu.async_copy(dat_v.at[pl.ds(j * cg, cg)],
                                 out_hbm.at[pl.ds(wid * RPW + j * cg, cg)],
                                 osem))
        for o in outs:
            o.wait()

    return body(fm_rows, pos2d)


def kernel(feature_map, vertices_positions):
    fm_rows = (
        feature_map.transpose(0, 2, 3, 1)
        .reshape(B, H, W // 8, 8, C // 128, 128)
        .transpose(0, 1, 2, 4, 3, 5)
        .reshape(B * H * (W // 8) * (C // 128) * 8, 128)
    )
    pos16 = (
        vertices_positions.astype(jnp.int32)
        .transpose(0, 2, 1)
        .reshape(B, 2, N // 128, 128)
        .transpose(0, 2, 1, 3)
        .reshape(B * 2 * N)
    )
    out = _sc_gather(fm_rows, pos16)
    return out.reshape(B, N, C)


# speedup vs baseline: 1.0030x; 1.0030x over previous
import functools

import jax
import jax.numpy as jnp
from jax import lax
from jax.experimental import pallas as pl
from jax.experimental.pallas import tpu as pltpu
from jax.experimental.pallas import tpu_sc as plsc

B, C, H, W = 2, 256, 320, 320
N = 512
NV = B * N
NWORK = 32
VPW = NV // NWORK
RPW = 2 * VPW
NROWS = NV * (C // 128)


def _sc_gather(fm_rows, pos2d):
    mesh = plsc.VectorSubcoreMesh(core_axis_name="c", subcore_axis_name="s")

    @functools.partial(
        pl.kernel,
        out_type=jax.ShapeDtypeStruct((NROWS, 128), jnp.float32),
        mesh=mesh,
        scratch_types=[
            pltpu.VMEM((160,), jnp.int32),
            pltpu.VMEM((RPW,), jnp.int32),
            pltpu.VMEM((RPW, 128), jnp.float32),
            pltpu.SemaphoreType.DMA,
            pltpu.SemaphoreType.DMA,
            pltpu.SemaphoreType.DMA,
        ],
        compiler_params=pltpu.CompilerParams(
            needs_layout_passes=False,
            skip_device_barrier=True,
            disable_bounds_checks=True,
            disable_semaphore_checks=True,
        ),
    )
    def body(fm_hbm, pos_hbm, out_hbm, pos_v, idx_v, dat_v, gsem0, gsem1,
             osem):
        wid = lax.axis_index("s") * 2 + lax.axis_index("c")
        v0 = wid * VPW
        bq = lax.shift_right_logical(v0, 9)
        k0 = lax.shift_right_logical(v0 & 511, 7)
        o = v0 & 127
        base = pl.multiple_of((bq * 8 + k0 * 2) * 128 + o, VPW)
        pltpu.sync_copy(pos_hbm.at[pl.ds(base, 160)], pos_v)

        lane = jax.lax.iota(jnp.int32, 16)
        hg = RPW // 2
        sems = (gsem0, gsem1)
        gathers = []
        for vc in range(VPW // 16):
            v_loc = vc * 16 + lane
            r = pos_v[pl.ds(vc * 16, 16)]
            c = pos_v[pl.ds(128 + vc * 16, 16)]
            b = lax.shift_right_logical(v0 + vc * 16 + lane, 9)
            rho0 = ((b * H + r) * (W // 8) + lax.shift_right_logical(c, 3)) \
                * 16 + (c & 7)
            pos = v_loc * 2
            plsc.store_scatter(idx_v, [pos], rho0)
            plsc.store_scatter(idx_v, [pos + 1], rho0 + 8)
            gathers.append(
                pltpu.async_copy(fm_hbm.at[idx_v.at[pl.ds(vc * hg, hg)]],
                                 dat_v.at[pl.ds(vc * hg, hg)], sems[vc]))

        outs = []
        for vc in range(VPW // 16):
            gathers[vc].wait()
            outs.append(
                pltpu.async_copy(dat_v.at[pl.ds(vc * hg, hg)],
                                 out_hbm.at[pl.ds(wid * RPW + vc * hg, hg)],
                                 osem))
        for o in outs:
            o.wait()

    return body(fm_rows, pos2d)


def kernel(feature_map, vertices_positions):
    fm_rows = (
        feature_map.transpose(0, 2, 3, 1)
        .reshape(B, H, W // 8, 8, C // 128, 128)
        .transpose(0, 1, 2, 4, 3, 5)
        .reshape(B * H * (W // 8) * (C // 128) * 8, 128)
    )
    pos16 = (
        vertices_positions.astype(jnp.int32)
        .transpose(0, 2, 1)
        .reshape(B, 2, N // 128, 128)
        .transpose(0, 2, 1, 3)
        .reshape(B * 2 * N)
    )
    out = _sc_gather(fm_rows, pos16)
    return out.reshape(B, N, C)


# speedup vs baseline: 1.0362x; 1.0331x over previous
import functools

import jax
import jax.numpy as jnp
from jax import lax
from jax.experimental import pallas as pl
from jax.experimental.pallas import tpu as pltpu
from jax.experimental.pallas import tpu_sc as plsc

B, C, H, W = 2, 256, 320, 320
N = 512
NV = B * N
NWORK = 16
VPW = NV // NWORK
RPW = 2 * VPW
NROWS = NV * (C // 128)
NG = VPW // 16
GR = RPW // NG


def _sc_gather(fm_rows, pos_flat):
    mesh = plsc.VectorSubcoreMesh(
        core_axis_name="c", subcore_axis_name="s", num_cores=1)

    @functools.partial(
        pl.kernel,
        out_type=jax.ShapeDtypeStruct((NROWS, 128), jnp.float32),
        mesh=mesh,
        scratch_types=[
            pltpu.VMEM((128 + VPW,), jnp.int32),
            pltpu.VMEM((RPW,), jnp.int32),
            pltpu.VMEM((RPW, 128), jnp.float32),
            [pltpu.SemaphoreType.DMA] * NG,
            pltpu.SemaphoreType.DMA,
        ],
        compiler_params=pltpu.CompilerParams(
            needs_layout_passes=False,
            skip_device_barrier=True,
            disable_bounds_checks=True,
            disable_semaphore_checks=True,
        ),
    )
    def body(fm_hbm, pos_hbm, out_hbm, pos_v, idx_v, dat_v, gsems, osem):
        wid = lax.axis_index("s") + lax.axis_index("c")
        v0 = wid * VPW
        bq = lax.shift_right_logical(v0, 9)
        k0 = lax.shift_right_logical(v0 & 511, 7)
        o = v0 & 127
        base = pl.multiple_of((bq * 8 + k0 * 2) * 128 + o, VPW)
        pltpu.sync_copy(pos_hbm.at[pl.ds(base, 128 + VPW)], pos_v)

        lane = jax.lax.iota(jnp.int32, 16)
        gathers = []
        for vc in range(NG):
            v_loc = vc * 16 + lane
            r = pos_v[pl.ds(vc * 16, 16)]
            c = pos_v[pl.ds(128 + vc * 16, 16)]
            b = lax.shift_right_logical(v0 + vc * 16 + lane, 9)
            rho0 = ((b * H + r) * (W // 8) + lax.shift_right_logical(c, 3)) \
                * 16 + (c & 7)
            pos = v_loc * 2
            plsc.store_scatter(idx_v, [pos], rho0)
            plsc.store_scatter(idx_v, [pos + 1], rho0 + 8)
            gathers.append(
                pltpu.async_copy(fm_hbm.at[idx_v.at[pl.ds(vc * GR, GR)]],
                                 dat_v.at[pl.ds(vc * GR, GR)], gsems[vc]))

        outs = []
        for vc in range(NG):
            gathers[vc].wait()
            outs.append(
                pltpu.async_copy(dat_v.at[pl.ds(vc * GR, GR)],
                                 out_hbm.at[pl.ds(wid * RPW + vc * GR, GR)],
                                 osem))
        for od in outs:
            od.wait()

    return body(fm_rows, pos_flat)


def kernel(feature_map, vertices_positions):
    fm_rows = (
        feature_map.transpose(0, 2, 3, 1)
        .reshape(B, H, W // 8, 8, C // 128, 128)
        .transpose(0, 1, 2, 4, 3, 5)
        .reshape(B * H * (W // 8) * (C // 128) * 8, 128)
    )
    pos_flat = (
        vertices_positions.astype(jnp.int32)
        .transpose(0, 2, 1)
        .reshape(B, 2, N // 128, 128)
        .transpose(0, 2, 1, 3)
        .reshape(B * 2 * N)
    )
    out = _sc_gather(fm_rows, pos_flat)
    return out.reshape(B, N, C)


# speedup vs baseline: 1.0442x; 1.0078x over previous
import functools

import jax
import jax.numpy as jnp
from jax import lax
from jax.experimental import pallas as pl
from jax.experimental.pallas import tpu as pltpu
from jax.experimental.pallas import tpu_sc as plsc

B, C, H, W = 2, 256, 320, 320
N = 512
NV = B * N
NWORK = 16
VPW = NV // NWORK
RPW = 2 * VPW
NROWS = NV * (C // 128)
NG = VPW // 16
GR = RPW // NG


def _sc_gather(fm_rows, pos_flat):
    mesh = plsc.VectorSubcoreMesh(
        core_axis_name="c", subcore_axis_name="s", num_cores=1)

    @functools.partial(
        pl.kernel,
        out_type=jax.ShapeDtypeStruct((NROWS, 128), jnp.float32),
        mesh=mesh,
        scratch_types=[
            pltpu.VMEM((128 + VPW,), jnp.int32),
            pltpu.VMEM((RPW,), jnp.int32),
            pltpu.VMEM((RPW, 128), jnp.float32),
            [pltpu.SemaphoreType.DMA] * NG,
            pltpu.SemaphoreType.DMA,
        ],
        compiler_params=pltpu.CompilerParams(
            needs_layout_passes=False,
            skip_device_barrier=True,
            disable_bounds_checks=True,
            disable_semaphore_checks=True,
        ),
    )
    def body(fm_hbm, pos_hbm, out_hbm, pos_v, idx_v, dat_v, gsems, osem):
        wid = lax.axis_index("s") + lax.axis_index("c")
        v0 = wid * VPW
        bq = lax.shift_right_logical(v0, 9)
        k0 = lax.shift_right_logical(v0 & 511, 7)
        o = v0 & 127
        base = pl.multiple_of((bq * 8 + k0 * 2) * 128 + o, VPW)
        pltpu.sync_copy(pos_hbm.at[pl.ds(base, 128 + VPW)], pos_v)

        lane = jax.lax.iota(jnp.int32, 16)
        gathers = []
        for vc in range(NG):
            v_loc = vc * 16 + lane
            r = pos_v[pl.ds(vc * 16, 16)]
            c = pos_v[pl.ds(128 + vc * 16, 16)]
            b = lax.shift_right_logical(v0 + vc * 16 + lane, 9)
            rho0 = ((b * H + r) * (W // 8) + lax.shift_right_logical(c, 3)) \
                * 16 + (c & 7)
            pos = v_loc * 2
            plsc.store_scatter(idx_v, [pos], rho0)
            plsc.store_scatter(idx_v, [pos + 1], rho0 + 8)
            if vc % 2 == 1:
                j = vc // 2
                hg = 2 * GR
                gathers.append(pltpu.async_copy(
                    fm_hbm.at[idx_v.at[pl.ds(j * hg, hg)]],
                    dat_v.at[pl.ds(j * hg, hg)], gsems[j]))

        outs = []
        for j in range(NG // 2):
            hg = 2 * GR
            gathers[j].wait()
            outs.append(
                pltpu.async_copy(dat_v.at[pl.ds(j * hg, hg)],
                                 out_hbm.at[pl.ds(wid * RPW + j * hg, hg)],
                                 osem))
        for od in outs:
            od.wait()

    return body(fm_rows, pos_flat)


def kernel(feature_map, vertices_positions):
    fm_rows = (
        feature_map.transpose(0, 2, 3, 1)
        .reshape(B, H, W // 8, 8, C // 128, 128)
        .transpose(0, 1, 2, 4, 3, 5)
        .reshape(B * H * (W // 8) * (C // 128) * 8, 128)
    )
    pos_flat = (
        vertices_positions.astype(jnp.int32)
        .transpose(0, 2, 1)
        .reshape(B, 2, N // 128, 128)
        .transpose(0, 2, 1, 3)
        .reshape(B * 2 * N)
    )
    out = _sc_gather(fm_rows, pos_flat)
    return out.reshape(B, N, C)
